# baseline (speedup 1.0000x reference)
import functools

import jax
import jax.numpy as jnp
from jax import lax
from jax.experimental import pallas as pl
from jax.experimental.pallas import tpu as pltpu
from jax.experimental.pallas import tpu_sc as plsc

EMBED = 64
VOCAB = 1024
LANES = 16
CHUNK = 128
TOK_PER_STEP = 128
IDX_PER_STEP = 4 * CHUNK
MLP_ROWS = 8192


def _tc_fold(table, W1):
    def body(t_ref, w1_ref, out_ref):
        out_ref[...] = jnp.dot(t_ref[...], w1_ref[...],
                               preferred_element_type=jnp.float32,
                               precision=lax.Precision.HIGHEST
                               ).astype(jnp.bfloat16)

    return pl.pallas_call(
        body,
        grid=(4,),
        in_specs=[
            pl.BlockSpec((VOCAB, EMBED), lambda c: (0, 0)),
            pl.BlockSpec((EMBED, EMBED), lambda c: (c, 0)),
        ],
        out_specs=pl.BlockSpec((VOCAB, EMBED), lambda c: (c, 0)),
        out_shape=jax.ShapeDtypeStruct((4 * VOCAB, EMBED), jnp.bfloat16),
    )(table, W1)


def _sc_gather_sum(tcat, idx):
    n_rows = idx.shape[0]
    n_tok = n_rows * TOK_PER_STEP
    mesh = plsc.VectorSubcoreMesh(core_axis_name="c", subcore_axis_name="s")

    @functools.partial(
        pl.kernel,
        out_type=jax.ShapeDtypeStruct((n_tok, EMBED), jnp.bfloat16),
        mesh=mesh,
        scratch_types=[
            pltpu.VMEM((4, CHUNK), jnp.int32),
            pltpu.VMEM((1, CHUNK), jnp.int32),
            pltpu.VMEM((IDX_PER_STEP, EMBED), jnp.bfloat16),
            pltpu.VMEM((TOK_PER_STEP, EMBED), jnp.bfloat16),
            pltpu.VMEM_SHARED((16 * TOK_PER_STEP, EMBED), jnp.bfloat16),
        ],
        compiler_params=pltpu.CompilerParams(use_tc_tiling_on_sc=False),
    )
    def gather_kernel(tcat_hbm, idx_hbm, out_hbm, gidx, tokidx, rows, zeros,
                      acc_shared):
        sid = lax.axis_index("s")
        acc_base = sid * TOK_PER_STEP

        @pl.loop(0, TOK_PER_STEP)
        def _(r):
            @pl.loop(0, EMBED // (2 * LANES))
            def _(k):
                zeros[pl.ds(r, 1), pl.ds(k * 2 * LANES, 2 * LANES)] = (
                    jnp.zeros((1, 2 * LANES), jnp.bfloat16))

        @pl.loop(0, CHUNK // LANES)
        def _(k):
            tid = lax.iota(jnp.int32, LANES) + (k * LANES + acc_base)
            tokidx[pl.ds(0, 1), pl.ds(k * LANES, LANES)] = tid.reshape(1, LANES)

        def body(i_vmem, o_vmem):
            for j in range(4):
                @pl.loop(0, CHUNK // LANES)
                def _(k, j=j):
                    raw = i_vmem[pl.ds(0, 1), pl.ds(j * CHUNK + k * LANES, LANES)]
                    gidx[pl.ds(j, 1), pl.ds(k * LANES, LANES)] = raw + (j << 10)

            pltpu.sync_copy(zeros, acc_shared.at[pl.ds(acc_base, TOK_PER_STEP)])
            for j in range(4):
                pltpu.sync_copy(tcat_hbm.at[gidx.at[j]],
                                rows.at[pl.ds(j * CHUNK, CHUNK)])
            for j in range(4):
                pltpu.sync_copy(rows.at[pl.ds(j * CHUNK, CHUNK)],
                                acc_shared.at[tokidx.at[0]], add=True)
            pltpu.sync_copy(acc_shared.at[pl.ds(acc_base, TOK_PER_STEP)], o_vmem)

        pltpu.emit_pipeline(
            body,
            grid=(n_rows,),
            in_specs=[pl.BlockSpec((1, IDX_PER_STEP), lambda i: (i, 0))],
            out_specs=[pl.BlockSpec((TOK_PER_STEP, EMBED), lambda i: (i, 0))],
            core_axis_name=("c", "s"),
            dimension_semantics=(pltpu.PARALLEL,),
        )(idx_hbm, out_hbm)

    return gather_kernel(tcat, idx)


def _tc_mlp_paired(h2, b1p, W2d, b2p):
    n2 = h2.shape[0]

    def body(h_ref, b1_ref, w2_ref, b2_ref, out_ref):
        h = h_ref[...].astype(jnp.float32) + b1_ref[...]
        h = h * jax.nn.sigmoid(h)
        out_ref[...] = jnp.dot(h, w2_ref[...],
                               preferred_element_type=jnp.float32) + b2_ref[...]

    return pl.pallas_call(
        body,
        grid=(n2 // MLP_ROWS,),
        in_specs=[
            pl.BlockSpec((MLP_ROWS, 2 * EMBED), lambda i: (i, 0)),
            pl.BlockSpec((1, 2 * EMBED), lambda i: (0, 0)),
            pl.BlockSpec((2 * EMBED, 2 * EMBED), lambda i: (0, 0)),
            pl.BlockSpec((1, 2 * EMBED), lambda i: (0, 0)),
        ],
        out_specs=pl.BlockSpec((MLP_ROWS, 2 * EMBED), lambda i: (i, 0)),
        out_shape=jax.ShapeDtypeStruct((n2, 2 * EMBED), jnp.float32),
    )(h2, b1p, W2d, b2p)


def kernel(x, table, W1, b1, W2, b2):
    bsz, seq, c = x.shape
    n = bsz * seq
    idx = (x.astype(jnp.int32)
           .reshape(bsz // CHUNK, CHUNK, seq, c)
           .transpose(2, 0, 3, 1)
           .reshape(seq * (bsz // CHUNK), c * CHUNK))
    tcat = _tc_fold(table, W1)
    h_pre = _sc_gather_sum(tcat, idx)
    h2 = h_pre.reshape(n // 2, 2 * EMBED)
    z = jnp.zeros((EMBED, EMBED), jnp.float32)
    W2d = jnp.block([[W2, z], [z, W2]])
    b1p = jnp.tile(b1, 2).reshape(1, -1)
    b2p = jnp.tile(b2, 2).reshape(1, -1)
    out2 = _tc_mlp_paired(h2, b1p, W2d, b2p)
    return out2.reshape(seq, bsz, EMBED).transpose(1, 0, 2)


# speedup vs baseline: 1.2401x; 1.2401x over previous
import functools

import jax
import jax.numpy as jnp
from jax import lax
from jax.experimental import pallas as pl
from jax.experimental.pallas import tpu as pltpu
from jax.experimental.pallas import tpu_sc as plsc

EMBED = 64
VOCAB = 1024
LANES = 16
CHUNK = 128
TOK_PER_STEP = 128
IDX_PER_STEP = 4 * CHUNK
MLP_ROWS = 8192


def _tc_fold(table, W1):
    def body(t_ref, w1_ref, out_ref):
        out_ref[...] = jnp.dot(t_ref[...], w1_ref[...],
                               preferred_element_type=jnp.float32,
                               precision=lax.Precision.HIGHEST
                               ).astype(jnp.bfloat16)

    return pl.pallas_call(
        body,
        grid=(4,),
        in_specs=[
            pl.BlockSpec((VOCAB, EMBED), lambda c: (0, 0)),
            pl.BlockSpec((EMBED, EMBED), lambda c: (c, 0)),
        ],
        out_specs=pl.BlockSpec((VOCAB, EMBED), lambda c: (c, 0)),
        out_shape=jax.ShapeDtypeStruct((4 * VOCAB, EMBED), jnp.bfloat16),
    )(table, W1)


def _sc_gather_sum(tcat, idx):
    n_rows = idx.shape[0]
    n_tok = n_rows * TOK_PER_STEP
    mesh = plsc.VectorSubcoreMesh(core_axis_name="c", subcore_axis_name="s")

    @functools.partial(
        pl.kernel,
        out_type=jax.ShapeDtypeStruct((n_tok, EMBED), jnp.bfloat16),
        mesh=mesh,
        scratch_types=[
            pltpu.VMEM((4, CHUNK), jnp.int32),
            pltpu.VMEM((1, CHUNK), jnp.int32),
            pltpu.VMEM((IDX_PER_STEP, EMBED), jnp.bfloat16),
            pltpu.VMEM((TOK_PER_STEP, EMBED), jnp.bfloat16),
            pltpu.VMEM_SHARED((16 * TOK_PER_STEP, EMBED), jnp.bfloat16),
            pltpu.SemaphoreType.DMA,
        ],
        compiler_params=pltpu.CompilerParams(use_tc_tiling_on_sc=False),
    )
    def gather_kernel(tcat_hbm, idx_hbm, out_hbm, gidx, tokidx, rows, zeros,
                      acc_shared, sem):
        sid = lax.axis_index("s")
        acc_base = sid * TOK_PER_STEP

        @pl.loop(0, TOK_PER_STEP)
        def _(r):
            @pl.loop(0, EMBED // (2 * LANES))
            def _(k):
                zeros[pl.ds(r, 1), pl.ds(k * 2 * LANES, 2 * LANES)] = (
                    jnp.zeros((1, 2 * LANES), jnp.bfloat16))

        @pl.loop(0, CHUNK // LANES)
        def _(k):
            tid = lax.iota(jnp.int32, LANES) + (k * LANES + acc_base)
            tokidx[pl.ds(0, 1), pl.ds(k * LANES, LANES)] = tid.reshape(1, LANES)

        def body(i_vmem, o_vmem):
            for j in range(4):
                @pl.loop(0, CHUNK // LANES)
                def _(k, j=j):
                    raw = i_vmem[pl.ds(0, 1), pl.ds(j * CHUNK + k * LANES, LANES)]
                    gidx[pl.ds(j, 1), pl.ds(k * LANES, LANES)] = raw + (j << 10)

            gathers = [
                pltpu.async_copy(tcat_hbm.at[gidx.at[j]],
                                 rows.at[pl.ds(j * CHUNK, CHUNK)], sem)
                for j in range(4)
            ]
            pltpu.sync_copy(zeros, acc_shared.at[pl.ds(acc_base, TOK_PER_STEP)])
            for cp in gathers:
                cp.wait()
            for j in range(4):
                pltpu.sync_copy(rows.at[pl.ds(j * CHUNK, CHUNK)],
                                acc_shared.at[tokidx.at[0]], add=True)
            pltpu.sync_copy(acc_shared.at[pl.ds(acc_base, TOK_PER_STEP)], o_vmem)

        pltpu.emit_pipeline(
            body,
            grid=(n_rows,),
            in_specs=[pl.BlockSpec((1, IDX_PER_STEP), lambda i: (i, 0))],
            out_specs=[pl.BlockSpec((TOK_PER_STEP, EMBED), lambda i: (i, 0))],
            core_axis_name=("c", "s"),
            dimension_semantics=(pltpu.PARALLEL,),
        )(idx_hbm, out_hbm)

    return gather_kernel(tcat, idx)


def _tc_mlp_paired(h2, b1p, W2d, b2p):
    n2 = h2.shape[0]

    def body(h_ref, b1_ref, w2_ref, b2_ref, out_ref):
        h = h_ref[...].astype(jnp.float32) + b1_ref[...]
        h = h * jax.nn.sigmoid(h)
        out_ref[...] = jnp.dot(h, w2_ref[...],
                               preferred_element_type=jnp.float32) + b2_ref[...]

    return pl.pallas_call(
        body,
        grid=(n2 // MLP_ROWS,),
        in_specs=[
            pl.BlockSpec((MLP_ROWS, 2 * EMBED), lambda i: (i, 0)),
            pl.BlockSpec((1, 2 * EMBED), lambda i: (0, 0)),
            pl.BlockSpec((2 * EMBED, 2 * EMBED), lambda i: (0, 0)),
            pl.BlockSpec((1, 2 * EMBED), lambda i: (0, 0)),
        ],
        out_specs=pl.BlockSpec((MLP_ROWS, 2 * EMBED), lambda i: (i, 0)),
        out_shape=jax.ShapeDtypeStruct((n2, 2 * EMBED), jnp.float32),
    )(h2, b1p, W2d, b2p)


def kernel(x, table, W1, b1, W2, b2):
    bsz, seq, c = x.shape
    n = bsz * seq
    idx = (x.astype(jnp.int32)
           .reshape(bsz // CHUNK, CHUNK, seq, c)
           .transpose(2, 0, 3, 1)
           .reshape(seq * (bsz // CHUNK), c * CHUNK))
    tcat = _tc_fold(table, W1)
    h_pre = _sc_gather_sum(tcat, idx)
    h2 = h_pre.reshape(n // 2, 2 * EMBED)
    z = jnp.zeros((EMBED, EMBED), jnp.float32)
    W2d = jnp.block([[W2, z], [z, W2]])
    b1p = jnp.tile(b1, 2).reshape(1, -1)
    b2p = jnp.tile(b2, 2).reshape(1, -1)
    out2 = _tc_mlp_paired(h2, b1p, W2d, b2p)
    return out2.reshape(seq, bsz, EMBED).transpose(1, 0, 2)


# speedup vs baseline: 1.2764x; 1.0293x over previous
import functools

import jax
import jax.numpy as jnp
from jax import lax
from jax.experimental import pallas as pl
from jax.experimental.pallas import tpu as pltpu
from jax.experimental.pallas import tpu_sc as plsc

EMBED = 64
VOCAB = 1024
LANES = 16
CHUNK = 128
TOK_PER_STEP = 128
IDX_PER_STEP = 4 * CHUNK
MLP_ROWS = 8192


def _tc_fold(table, W1):
    def body(t_ref, w1_ref, out_ref):
        out_ref[...] = jnp.dot(t_ref[...], w1_ref[...],
                               preferred_element_type=jnp.float32,
                               precision=lax.Precision.HIGHEST
                               ).astype(jnp.bfloat16)

    return pl.pallas_call(
        body,
        grid=(4,),
        in_specs=[
            pl.BlockSpec((VOCAB, EMBED), lambda c: (0, 0)),
            pl.BlockSpec((EMBED, EMBED), lambda c: (c, 0)),
        ],
        out_specs=pl.BlockSpec((VOCAB, EMBED), lambda c: (c, 0)),
        out_shape=jax.ShapeDtypeStruct((4 * VOCAB, EMBED), jnp.bfloat16),
    )(table, W1)


def _sc_gather_sum(tcat, idx):
    n_rows = idx.shape[0]
    n_tok = n_rows * TOK_PER_STEP
    mesh = plsc.VectorSubcoreMesh(core_axis_name="c", subcore_axis_name="s")

    @functools.partial(
        pl.kernel,
        out_type=jax.ShapeDtypeStruct((n_tok, EMBED), jnp.bfloat16),
        mesh=mesh,
        scratch_types=[
            pltpu.VMEM((4, CHUNK), jnp.int32),
            pltpu.VMEM((1, CHUNK), jnp.int32),
            pltpu.VMEM((IDX_PER_STEP, EMBED), jnp.bfloat16),
            pltpu.VMEM((TOK_PER_STEP, EMBED), jnp.bfloat16),
            pltpu.VMEM_SHARED((16 * TOK_PER_STEP, EMBED), jnp.bfloat16),
            pltpu.SemaphoreType.DMA,
        ],
        compiler_params=pltpu.CompilerParams(use_tc_tiling_on_sc=False),
    )
    def gather_kernel(tcat_hbm, idx_hbm, out_hbm, gidx, tokidx, rows, zeros,
                      acc_shared, sem):
        sid = lax.axis_index("s")
        acc_base = sid * TOK_PER_STEP

        @pl.loop(0, TOK_PER_STEP)
        def _(r):
            @pl.loop(0, EMBED // (2 * LANES))
            def _(k):
                zeros[pl.ds(r, 1), pl.ds(k * 2 * LANES, 2 * LANES)] = (
                    jnp.zeros((1, 2 * LANES), jnp.bfloat16))

        @pl.loop(0, CHUNK // LANES)
        def _(k):
            tid = lax.iota(jnp.int32, LANES) + (k * LANES + acc_base)
            tokidx[pl.ds(0, 1), pl.ds(k * LANES, LANES)] = tid.reshape(1, LANES)

        def body(i_vmem, o_vmem):
            for j in range(4):
                @pl.loop(0, CHUNK // LANES)
                def _(k, j=j):
                    raw = i_vmem[pl.ds(0, 1), pl.ds(j * CHUNK + k * LANES, LANES)]
                    gidx[pl.ds(j, 1), pl.ds(k * LANES, LANES)] = raw + (j << 10)

            gathers = [
                pltpu.async_copy(tcat_hbm.at[gidx.at[j]],
                                 rows.at[pl.ds(j * CHUNK, CHUNK)], sem)
                for j in range(4)
            ]
            pltpu.sync_copy(zeros, acc_shared.at[pl.ds(acc_base, TOK_PER_STEP)])
            for cp in gathers:
                cp.wait()
            adds = [
                pltpu.async_copy(rows.at[pl.ds(j * CHUNK, CHUNK)],
                                 acc_shared.at[tokidx.at[0]], sem, add=True)
                for j in range(4)
            ]
            for cp in adds:
                cp.wait()
            pltpu.sync_copy(acc_shared.at[pl.ds(acc_base, TOK_PER_STEP)], o_vmem)

        pltpu.emit_pipeline(
            body,
            grid=(n_rows,),
            in_specs=[pl.BlockSpec((1, IDX_PER_STEP), lambda i: (i, 0))],
            out_specs=[pl.BlockSpec((TOK_PER_STEP, EMBED), lambda i: (i, 0))],
            core_axis_name=("c", "s"),
            dimension_semantics=(pltpu.PARALLEL,),
        )(idx_hbm, out_hbm)

    return gather_kernel(tcat, idx)


def _tc_mlp_paired(h2, b1p, W2d, b2p):
    n2 = h2.shape[0]

    def body(h_ref, b1_ref, w2_ref, b2_ref, out_ref):
        h = h_ref[...].astype(jnp.float32) + b1_ref[...]
        h = h * jax.nn.sigmoid(h)
        out_ref[...] = jnp.dot(h, w2_ref[...],
                               preferred_element_type=jnp.float32) + b2_ref[...]

    return pl.pallas_call(
        body,
        grid=(n2 // MLP_ROWS,),
        in_specs=[
            pl.BlockSpec((MLP_ROWS, 2 * EMBED), lambda i: (i, 0)),
            pl.BlockSpec((1, 2 * EMBED), lambda i: (0, 0)),
            pl.BlockSpec((2 * EMBED, 2 * EMBED), lambda i: (0, 0)),
            pl.BlockSpec((1, 2 * EMBED), lambda i: (0, 0)),
        ],
        out_specs=pl.BlockSpec((MLP_ROWS, 2 * EMBED), lambda i: (i, 0)),
        out_shape=jax.ShapeDtypeStruct((n2, 2 * EMBED), jnp.float32),
    )(h2, b1p, W2d, b2p)


def kernel(x, table, W1, b1, W2, b2):
    bsz, seq, c = x.shape
    n = bsz * seq
    idx = (x.astype(jnp.int32)
           .reshape(bsz // CHUNK, CHUNK, seq, c)
           .transpose(2, 0, 3, 1)
           .reshape(seq * (bsz // CHUNK), c * CHUNK))
    tcat = _tc_fold(table, W1)
    h_pre = _sc_gather_sum(tcat, idx)
    h2 = h_pre.reshape(n // 2, 2 * EMBED)
    z = jnp.zeros((EMBED, EMBED), jnp.float32)
    W2d = jnp.block([[W2, z], [z, W2]])
    b1p = jnp.tile(b1, 2).reshape(1, -1)
    b2p = jnp.tile(b2, 2).reshape(1, -1)
    out2 = _tc_mlp_paired(h2, b1p, W2d, b2p)
    return out2.reshape(seq, bsz, EMBED).transpose(1, 0, 2)


# speedup vs baseline: 1.3303x; 1.0422x over previous
import functools

import jax
import jax.numpy as jnp
from jax import lax
from jax.experimental import pallas as pl
from jax.experimental.pallas import tpu as pltpu
from jax.experimental.pallas import tpu_sc as plsc

EMBED = 64
VOCAB = 1024
LANES = 16
CHUNK = 128
TOK_PER_STEP = 256
IDX_PER_STEP = 4 * TOK_PER_STEP
N_CHUNKS = IDX_PER_STEP // CHUNK
MLP_ROWS = 8192


def _tc_fold(table, W1):
    def body(t_ref, w1_ref, out_ref):
        out_ref[...] = jnp.dot(t_ref[...], w1_ref[...],
                               preferred_element_type=jnp.float32,
                               precision=lax.Precision.HIGHEST
                               ).astype(jnp.bfloat16)

    return pl.pallas_call(
        body,
        grid=(4,),
        in_specs=[
            pl.BlockSpec((VOCAB, EMBED), lambda c: (0, 0)),
            pl.BlockSpec((EMBED, EMBED), lambda c: (c, 0)),
        ],
        out_specs=pl.BlockSpec((VOCAB, EMBED), lambda c: (c, 0)),
        out_shape=jax.ShapeDtypeStruct((4 * VOCAB, EMBED), jnp.bfloat16),
    )(table, W1)


def _sc_gather_sum(tcat, idx):
    n_rows = idx.shape[0]
    n_tok = n_rows * TOK_PER_STEP
    mesh = plsc.VectorSubcoreMesh(core_axis_name="c", subcore_axis_name="s")

    @functools.partial(
        pl.kernel,
        out_type=jax.ShapeDtypeStruct((n_tok, EMBED), jnp.bfloat16),
        mesh=mesh,
        scratch_types=[
            pltpu.VMEM((N_CHUNKS, CHUNK), jnp.int32),
            pltpu.VMEM((2, CHUNK), jnp.int32),
            pltpu.VMEM((IDX_PER_STEP, EMBED), jnp.bfloat16),
            pltpu.VMEM((TOK_PER_STEP, EMBED), jnp.bfloat16),
            pltpu.VMEM_SHARED((16 * TOK_PER_STEP, EMBED), jnp.bfloat16),
            pltpu.SemaphoreType.DMA,
        ],
        compiler_params=pltpu.CompilerParams(use_tc_tiling_on_sc=False),
    )
    def gather_kernel(tcat_hbm, idx_hbm, out_hbm, gidx, tokidx, rows, zeros,
                      acc_shared, sem):
        sid = lax.axis_index("s")
        acc_base = sid * TOK_PER_STEP

        @pl.loop(0, TOK_PER_STEP)
        def _(r):
            @pl.loop(0, EMBED // (2 * LANES))
            def _(k):
                zeros[pl.ds(r, 1), pl.ds(k * 2 * LANES, 2 * LANES)] = (
                    jnp.zeros((1, 2 * LANES), jnp.bfloat16))

        for h in range(2):
            @pl.loop(0, CHUNK // LANES)
            def _(k, h=h):
                tid = lax.iota(jnp.int32, LANES) + (h * CHUNK + k * LANES + acc_base)
                tokidx[pl.ds(h, 1), pl.ds(k * LANES, LANES)] = (
                    tid.reshape(1, LANES))

        def body(i_vmem, o_vmem):
            for q in range(N_CHUNKS):
                @pl.loop(0, CHUNK // LANES)
                def _(k, q=q):
                    raw = i_vmem[pl.ds(0, 1), pl.ds(q * CHUNK + k * LANES, LANES)]
                    gidx[pl.ds(q, 1), pl.ds(k * LANES, LANES)] = (
                        raw + ((q % 4) << 10))

            gathers = [
                pltpu.async_copy(tcat_hbm.at[gidx.at[q]],
                                 rows.at[pl.ds(q * CHUNK, CHUNK)], sem)
                for q in range(N_CHUNKS)
            ]
            pltpu.sync_copy(zeros, acc_shared.at[pl.ds(acc_base, TOK_PER_STEP)])
            for cp in gathers:
                cp.wait()
            adds = [
                pltpu.async_copy(rows.at[pl.ds(q * CHUNK, CHUNK)],
                                 acc_shared.at[tokidx.at[q // 4]], sem, add=True)
                for q in range(N_CHUNKS)
            ]
            for cp in adds:
                cp.wait()
            pltpu.sync_copy(acc_shared.at[pl.ds(acc_base, TOK_PER_STEP)], o_vmem)

        pltpu.emit_pipeline(
            body,
            grid=(n_rows,),
            in_specs=[pl.BlockSpec((1, IDX_PER_STEP), lambda i: (i, 0))],
            out_specs=[pl.BlockSpec((TOK_PER_STEP, EMBED), lambda i: (i, 0))],
            core_axis_name=("c", "s"),
            dimension_semantics=(pltpu.PARALLEL,),
        )(idx_hbm, out_hbm)

    return gather_kernel(tcat, idx)


def _tc_mlp_paired(h2, b1p, W2d, b2p):
    n2 = h2.shape[0]

    def body(h_ref, b1_ref, w2_ref, b2_ref, out_ref):
        h = h_ref[...].astype(jnp.float32) + b1_ref[...]
        h = h * jax.nn.sigmoid(h)
        out_ref[...] = jnp.dot(h, w2_ref[...],
                               preferred_element_type=jnp.float32) + b2_ref[...]

    return pl.pallas_call(
        body,
        grid=(n2 // MLP_ROWS,),
        in_specs=[
            pl.BlockSpec((MLP_ROWS, 2 * EMBED), lambda i: (i, 0)),
            pl.BlockSpec((1, 2 * EMBED), lambda i: (0, 0)),
            pl.BlockSpec((2 * EMBED, 2 * EMBED), lambda i: (0, 0)),
            pl.BlockSpec((1, 2 * EMBED), lambda i: (0, 0)),
        ],
        out_specs=pl.BlockSpec((MLP_ROWS, 2 * EMBED), lambda i: (i, 0)),
        out_shape=jax.ShapeDtypeStruct((n2, 2 * EMBED), jnp.float32),
    )(h2, b1p, W2d, b2p)


def kernel(x, table, W1, b1, W2, b2):
    bsz, seq, c = x.shape
    n = bsz * seq
    idx = (x.astype(jnp.int32)
           .reshape(bsz // CHUNK, CHUNK, seq, c)
           .transpose(2, 0, 3, 1)
           .reshape(seq * bsz // TOK_PER_STEP, 4 * TOK_PER_STEP))
    tcat = _tc_fold(table, W1)
    h_pre = _sc_gather_sum(tcat, idx)
    h2 = h_pre.reshape(n // 2, 2 * EMBED)
    z = jnp.zeros((EMBED, EMBED), jnp.float32)
    W2d = jnp.block([[W2, z], [z, W2]])
    b1p = jnp.tile(b1, 2).reshape(1, -1)
    b2p = jnp.tile(b2, 2).reshape(1, -1)
    out2 = _tc_mlp_paired(h2, b1p, W2d, b2p)
    return out2.reshape(seq, bsz, EMBED).transpose(1, 0, 2)
